# baseline (speedup 1.0000x reference)
import functools
import jax
import jax.numpy as jnp
from jax import lax
from jax.experimental import pallas as pl
from jax.experimental.pallas import tpu as pltpu
from jax.experimental.pallas import tpu_sc as plsc

VOCAB = 1000000
EMBED_DIM = 64
BATCH = 4096
HIST = 50

NUM_CORES = 2
NUM_SUBCORES = 16
NW = NUM_CORES * NUM_SUBCORES
BBLK = BATCH // NW

_mesh = plsc.VectorSubcoreMesh(
    core_axis_name="c", subcore_axis_name="s",
    num_cores=NUM_CORES, num_subcores=NUM_SUBCORES,
)


@functools.partial(
    pl.kernel,
    mesh=_mesh,
    out_type=jax.ShapeDtypeStruct((HIST, 8, NW, 8, BBLK), jnp.float32),
    scratch_types=[
        pltpu.VMEM((HIST, BBLK), jnp.int32),
        [pltpu.VMEM((BBLK, EMBED_DIM), jnp.float32) for _ in range(2)],
        [pltpu.VMEM((8, 8, BBLK), jnp.float32) for _ in range(2)],
        [pltpu.SemaphoreType.DMA for _ in range(2)],
        [pltpu.SemaphoreType.DMA for _ in range(2)],
    ],
    compiler_params=pltpu.CompilerParams(use_tc_tiling_on_sc=False,
                                         needs_layout_passes=False),
)
def _embed_gather(idx_hbm, table_hbm, out_hbm, idx_v, abuf, bbuf, gsem, osem):
    wid = lax.axis_index("s") * NUM_CORES + lax.axis_index("c")

    pltpu.sync_copy(idx_hbm.at[:, wid], idx_v)

    iotas = [lax.iota(jnp.int32, 16) + 16 * k for k in range(8)]

    pltpu.async_copy(table_hbm.at[idx_v.at[0]], abuf[0], gsem[0])

    def step(h, pb):
        @pl.when(h + 1 < HIST)
        def _fire_gather():
            pltpu.async_copy(table_hbm.at[idx_v.at[h + 1]], abuf[1 - pb],
                             gsem[1 - pb])

        pltpu.make_async_copy(table_hbm.at[idx_v.at[h]], abuf[pb],
                              gsem[pb]).wait()

        @pl.when(h >= 2)
        def _drain_out():
            pltpu.make_async_copy(bbuf[pb], out_hbm.at[h - 2, :, wid],
                                  osem[pb]).wait()

        a, b = abuf[pb], bbuf[pb]

        @pl.loop(0, 8)
        def _ti(i):
            for s in range(8):
                c = jnp.full((16,), i * 8 + s, jnp.int32)
                for k in range(8):
                    b[i, s, pl.ds(16 * k, 16)] = plsc.load_gather(
                        a, [iotas[k], c])

        pltpu.async_copy(b, out_hbm.at[h, :, wid], osem[pb])

    @pl.loop(0, HIST, step=2)
    def _run(h0):
        step(h0, 0)
        step(h0 + 1, 1)

    for pb in range(2):
        pltpu.make_async_copy(bbuf[pb], out_hbm.at[HIST - 2 + pb, :, wid],
                              osem[pb]).wait()


def kernel(input, table):
    idx = input.astype(jnp.int32).T.reshape(HIST, NW, BBLK)
    out5 = _embed_gather(idx, table)
    return out5.transpose(2, 4, 0, 1, 3).reshape(BATCH, HIST, EMBED_DIM)


# speedup vs baseline: 1.1898x; 1.1898x over previous
import functools
import jax
import jax.numpy as jnp
from jax import lax
from jax.experimental import pallas as pl
from jax.experimental.pallas import tpu as pltpu
from jax.experimental.pallas import tpu_sc as plsc

VOCAB = 1000000
EMBED_DIM = 64
BATCH = 4096
HIST = 50
HPAD = 64

NUM_CORES = 2
NUM_SUBCORES = 16
NW = NUM_CORES * NUM_SUBCORES
BBLK = BATCH // NW

_mesh = plsc.VectorSubcoreMesh(
    core_axis_name="c", subcore_axis_name="s",
    num_cores=NUM_CORES, num_subcores=NUM_SUBCORES,
)


@functools.partial(
    pl.kernel,
    mesh=_mesh,
    out_type=jax.ShapeDtypeStruct((HIST, 8, NW, 8, BBLK), jnp.float32),
    scratch_types=[
        pltpu.VMEM((HPAD, BBLK), jnp.int32),
        pltpu.VMEM((HPAD, BBLK), jnp.int32),
        [pltpu.VMEM((BBLK, 128), jnp.float32) for _ in range(2)],
        [pltpu.VMEM((8, 8, BBLK), jnp.float32) for _ in range(2)],
        [pltpu.SemaphoreType.DMA for _ in range(2)],
        [pltpu.SemaphoreType.DMA for _ in range(2)],
    ],
    compiler_params=pltpu.CompilerParams(use_tc_tiling_on_sc=True,
                                         needs_layout_passes=False),
)
def _embed_gather(idxh_hbm, colb_hbm, table2_hbm, out_hbm,
                  idx_v, col_v, abuf, bbuf, gsem, osem):
    wid = lax.axis_index("s") * NUM_CORES + lax.axis_index("c")

    pltpu.sync_copy(idxh_hbm.at[wid], idx_v)
    pltpu.sync_copy(colb_hbm.at[wid], col_v)

    iotas = [lax.iota(jnp.int32, 16) + 16 * k for k in range(8)]

    pltpu.async_copy(table2_hbm.at[idx_v.at[0]], abuf[0], gsem[0])

    def step(h, pb):
        @pl.when(h + 1 < HIST)
        def _fire_gather():
            pltpu.async_copy(table2_hbm.at[idx_v.at[h + 1]], abuf[1 - pb],
                             gsem[1 - pb])

        pltpu.make_async_copy(table2_hbm.at[idx_v.at[h]], abuf[pb],
                              gsem[pb]).wait()

        @pl.when(h >= 2)
        def _drain_out():
            pltpu.make_async_copy(bbuf[pb], out_hbm.at[h - 2, :, wid],
                                  osem[pb]).wait()

        a, b = abuf[pb], bbuf[pb]
        cvec = [col_v[h, pl.ds(16 * k, 16)] for k in range(8)]

        @plsc.parallel_loop(0, EMBED_DIM, unroll=4)
        def _tc(c):
            i = lax.shift_right_logical(c, 2 + 1)
            s = lax.rem(c, 8)
            cc = jnp.full((16,), c, jnp.int32)
            for k in range(8):
                b[i, s, pl.ds(16 * k, 16)] = plsc.load_gather(
                    a, [iotas[k], cc + cvec[k]])

        pltpu.async_copy(b, out_hbm.at[h, :, wid], osem[pb])

    @pl.loop(0, HIST, step=2)
    def _run(h0):
        step(h0, 0)
        step(h0 + 1, 1)

    for pb in range(2):
        pltpu.make_async_copy(bbuf[pb], out_hbm.at[HIST - 2 + pb, :, wid],
                              osem[pb]).wait()


def kernel(input, table):
    per_w = input.astype(jnp.int32).T.reshape(HIST, NW, BBLK).transpose(1, 0, 2)
    pad = jnp.zeros((NW, HPAD - HIST, BBLK), jnp.int32)
    idx_half = jnp.concatenate([per_w >> 1, pad], axis=1)
    colbase = jnp.concatenate([(per_w & 1) << 6, pad], axis=1)
    table2 = table.reshape(VOCAB // 2, 2 * EMBED_DIM)
    out5 = _embed_gather(idx_half, colbase, table2)
    return out5.transpose(2, 4, 0, 1, 3).reshape(BATCH, HIST, EMBED_DIM)


# speedup vs baseline: 1.6353x; 1.3744x over previous
import functools
import jax
import jax.numpy as jnp
from jax import lax
from jax.experimental import pallas as pl
from jax.experimental.pallas import tpu as pltpu
from jax.experimental.pallas import tpu_sc as plsc

VOCAB = 1000000
EMBED_DIM = 64
BATCH = 4096
HIST = 50
HPAD = 64

NUM_CORES = 2
NUM_SUBCORES = 16
NW = NUM_CORES * NUM_SUBCORES
BBLK = BATCH // NW

_mesh = plsc.VectorSubcoreMesh(
    core_axis_name="c", subcore_axis_name="s",
    num_cores=NUM_CORES, num_subcores=NUM_SUBCORES,
)


@functools.partial(
    pl.kernel,
    mesh=_mesh,
    out_type=jax.ShapeDtypeStruct((HIST, 8, NW, 8, BBLK), jnp.float32),
    scratch_types=[
        pltpu.VMEM((2, BBLK), jnp.int32),
        [pltpu.VMEM((BBLK, EMBED_DIM), jnp.float32) for _ in range(2)],
        [pltpu.VMEM((8, 8, BBLK), jnp.float32) for _ in range(2)],
        [pltpu.SemaphoreType.DMA for _ in range(2)],
        [pltpu.SemaphoreType.DMA for _ in range(2)],
    ],
    compiler_params=pltpu.CompilerParams(use_tc_tiling_on_sc=True,
                                         needs_layout_passes=False),
)
def _embed_gather(idx_hbm, table_hbm, out_hbm,
                  idx_v, abuf, bbuf, gsem, osem):
    wid = lax.axis_index("s") * NUM_CORES + lax.axis_index("c")

    iotas = [lax.iota(jnp.int32, 16) + 16 * k for k in range(8)]

    def fire_gathers(h, pb):
        @pl.loop(0, BBLK, step=16)
        def _fire(k):
            vec = idx_v[pb, pl.ds(k, 16)]
            for j in range(16):
                pltpu.async_copy(table_hbm.at[vec[j]], abuf[pb].at[k + j],
                                 gsem[pb])

    def stage_idx(h, pb):
        pltpu.sync_copy(idx_hbm.at[wid, h], idx_v.at[pb])

    stage_idx(0, 0)
    fire_gathers(0, 0)

    def step(h, pb):
        @pl.when(h + 1 < HIST)
        def _fire_next():
            stage_idx(h + 1, 1 - pb)
            fire_gathers(h + 1, 1 - pb)

        pltpu.make_async_copy(table_hbm.at[pl.ds(0, BBLK)], abuf[pb],
                              gsem[pb]).wait()

        @pl.when(h >= 2)
        def _drain_out():
            pltpu.make_async_copy(bbuf[pb], out_hbm.at[h - 2, :, wid],
                                  osem[pb]).wait()

        a, b = abuf[pb], bbuf[pb]

        @plsc.parallel_loop(0, EMBED_DIM, unroll=4)
        def _tc(c):
            i = lax.shift_right_logical(c, 3)
            s = lax.rem(c, 8)
            cc = jnp.full((16,), c, jnp.int32)
            for k in range(8):
                b[i, s, pl.ds(16 * k, 16)] = plsc.load_gather(
                    a, [iotas[k], cc])

        pltpu.async_copy(b, out_hbm.at[h, :, wid], osem[pb])

    @pl.loop(0, HIST, step=2)
    def _run(h0):
        step(h0, 0)
        step(h0 + 1, 1)

    for pb in range(2):
        pltpu.make_async_copy(bbuf[pb], out_hbm.at[HIST - 2 + pb, :, wid],
                              osem[pb]).wait()


def kernel(input, table):
    per_w = input.astype(jnp.int32).T.reshape(HIST, NW, BBLK).transpose(1, 0, 2)
    pad = jnp.zeros((NW, HPAD - HIST, BBLK), jnp.int32)
    idx3 = jnp.concatenate([per_w, pad], axis=1)
    out5 = _embed_gather(idx3, table)
    return out5.transpose(2, 4, 0, 1, 3).reshape(BATCH, HIST, EMBED_DIM)


# speedup vs baseline: 1.6357x; 1.0003x over previous
import functools
import jax
import jax.numpy as jnp
from jax import lax
from jax.experimental import pallas as pl
from jax.experimental.pallas import tpu as pltpu
from jax.experimental.pallas import tpu_sc as plsc

VOCAB = 1000000
EMBED_DIM = 64
BATCH = 4096
HIST = 50
HPAD = 64

NUM_CORES = 2
NUM_SUBCORES = 16
NW = NUM_CORES * NUM_SUBCORES
BBLK = BATCH // NW

_mesh = plsc.VectorSubcoreMesh(
    core_axis_name="c", subcore_axis_name="s",
    num_cores=NUM_CORES, num_subcores=NUM_SUBCORES,
)


@functools.partial(
    pl.kernel,
    mesh=_mesh,
    out_type=jax.ShapeDtypeStruct((HIST, 8, NW, 8, BBLK), jnp.float32),
    scratch_types=[
        pltpu.VMEM((2, BBLK), jnp.int32),
        [pltpu.VMEM((BBLK, EMBED_DIM), jnp.float32) for _ in range(2)],
        [pltpu.VMEM((8, 8, BBLK), jnp.float32) for _ in range(2)],
        [pltpu.SemaphoreType.DMA for _ in range(2)],
        [pltpu.SemaphoreType.DMA for _ in range(2)],
    ],
    compiler_params=pltpu.CompilerParams(use_tc_tiling_on_sc=True,
                                         needs_layout_passes=False),
)
def _embed_gather(idx_hbm, table_hbm, out_hbm,
                  idx_v, abuf, bbuf, gsem, osem):
    wid = lax.axis_index("s") * NUM_CORES + lax.axis_index("c")

    iotas = [lax.iota(jnp.int32, 16) + 16 * k for k in range(8)]

    def fire_gathers(h, pb):
        @pl.loop(0, BBLK, step=16)
        def _fire(k):
            vec = idx_v[pb, pl.ds(k, 16)]
            for j in range(16):
                pltpu.async_copy(table_hbm.at[vec[j]], abuf[pb].at[k + j],
                                 gsem[pb])

    def stage_idx(h, pb):
        pltpu.sync_copy(idx_hbm.at[wid, h], idx_v.at[pb])

    stage_idx(0, 0)
    fire_gathers(0, 0)

    def step(h, pb):
        @pl.when(h + 1 < HIST)
        def _fire_next():
            stage_idx(h + 1, 1 - pb)
            fire_gathers(h + 1, 1 - pb)

        pltpu.make_async_copy(table_hbm.at[pl.ds(0, BBLK)], abuf[pb],
                              gsem[pb]).wait()

        @pl.when(h >= 2)
        def _drain_out():
            pltpu.make_async_copy(bbuf[pb], out_hbm.at[h - 2, :, wid],
                                  osem[pb]).wait()

        a, b = abuf[pb], bbuf[pb]

        @plsc.parallel_loop(0, EMBED_DIM, unroll=8)
        def _tc(c):
            i = lax.shift_right_logical(c, 3)
            s = lax.rem(c, 8)
            cc = jnp.full((16,), c, jnp.int32)
            for k in range(8):
                b[i, s, pl.ds(16 * k, 16)] = plsc.load_gather(
                    a, [iotas[k], cc])

        pltpu.async_copy(b, out_hbm.at[h, :, wid], osem[pb])

    @pl.loop(0, HIST, step=2)
    def _run(h0):
        step(h0, 0)
        step(h0 + 1, 1)

    for pb in range(2):
        pltpu.make_async_copy(bbuf[pb], out_hbm.at[HIST - 2 + pb, :, wid],
                              osem[pb]).wait()


def kernel(input, table):
    per_w = input.astype(jnp.int32).T.reshape(HIST, NW, BBLK).transpose(1, 0, 2)
    pad = jnp.zeros((NW, HPAD - HIST, BBLK), jnp.int32)
    idx3 = jnp.concatenate([per_w, pad], axis=1)
    out5 = _embed_gather(idx3, table)
    return out5.transpose(2, 4, 0, 1, 3).reshape(BATCH, HIST, EMBED_DIM)
